# Initial kernel scaffold; baseline (speedup 1.0000x reference)
import jax
import jax.numpy as jnp
from jax.experimental import pallas as pl


def kernel(X, edge_index, W1, att1_src, att1_dst, W2, att2_src, att2_dst):
    raise NotImplementedError("write your pallas kernel here")


# speedup vs baseline: 24.0960x; 24.0960x over previous
import functools

import jax
import jax.numpy as jnp
from jax import lax
from jax.experimental import pallas as pl
from jax.experimental.pallas import tpu as pltpu
from jax.experimental.pallas import tpu_sc as plsc

NW = 32
LANES = 16
KCH = 128


def _lrelu(t):
    return jnp.where(t > 0, t, 0.2 * t)


def _enc_body(x_ref, w_ref, ats_ref, atd_ref, he_ref, a_ref, b_ref, ps_ref):
    n = x_ref.shape[0]
    h = jnp.dot(x_ref[...], w_ref[...], preferred_element_type=jnp.float32)
    a = jnp.dot(h, ats_ref[...], preferred_element_type=jnp.float32)
    b = jnp.dot(h, atd_ref[...], preferred_element_type=jnp.float32)
    a_ref[...] = a
    b_ref[...] = b
    amax = jnp.max(a)
    sh = _lrelu(amax + b)
    ps_ref[...] = jnp.exp(_lrelu(a + b) - sh)
    ones_col = jnp.where(
        lax.broadcasted_iota(jnp.int32, (n, LANES), 1) == 0, 1.0, 0.0)
    he_ref[...] = jnp.concatenate([h, ones_col], axis=1)


def _dense_stage(x, W, att_s, att_d):
    n = x.shape[0]
    dout = W.shape[1]
    return pl.pallas_call(
        _enc_body,
        out_shape=[
            jax.ShapeDtypeStruct((n, dout + LANES), jnp.float32),
            jax.ShapeDtypeStruct((n, 1), jnp.float32),
            jax.ShapeDtypeStruct((n, 1), jnp.float32),
            jax.ShapeDtypeStruct((n, 1), jnp.float32),
        ],
    )(x, W, att_s.reshape(-1, 1), att_d.reshape(-1, 1))


def _norm_body(num_ref, ps_ref, he_ref, out_ref, *, dout, act):
    v = num_ref[...]
    ps = ps_ref[...]
    h = he_ref[...][:, :dout]
    num = v[0, :, :dout] + v[1, :, :dout] + ps * h
    den = v[0, :, dout:dout + 1] + v[1, :, dout:dout + 1] + ps + 1e-16
    y = num / den
    if act:
        y = jnp.where(y > 0, y, jnp.exp(y) - 1.0)
    out_ref[...] = y


def _normalize(num, ps, he, dout, act):
    n = ps.shape[0]
    return pl.pallas_call(
        functools.partial(_norm_body, dout=dout, act=act),
        out_shape=jax.ShapeDtypeStruct((n, dout), jnp.float32),
    )(num, ps, he)


def _edge_pass(n, d, epw, epw_pad):
    de = d + LANES
    nchunks = epw_pad // KCH
    rpt = (n // LANES) // 8 * 8
    rem_off = rpt * LANES
    rem = n - rem_off
    nz = rpt // 6
    mesh = plsc.VectorSubcoreMesh(core_axis_name="c", subcore_axis_name="s")

    @functools.partial(
        pl.kernel,
        mesh=mesh,
        compiler_params=pltpu.CompilerParams(
            needs_layout_passes=False, use_tc_tiling_on_sc=False),
        out_type=jax.ShapeDtypeStruct((2, n, de), jnp.float32),
        scratch_types=[
            pltpu.VMEM((n,), jnp.float32),
            pltpu.VMEM((n,), jnp.float32),
            pltpu.VMEM((KCH,), jnp.int32),
            pltpu.VMEM((KCH,), jnp.int32),
            pltpu.VMEM((KCH,), jnp.float32),
            pltpu.VMEM((KCH, de), jnp.float32),
            pltpu.VMEM_SHARED((n, de), jnp.float32),
            pltpu.SemaphoreType.DMA,
        ],
    )
    def edge_kernel(src_hbm, dst_hbm, a_hbm, b_hbm, he_hbm, out_hbm,
                    a_v, b_v, sidx, didx, p_v, rows, acc, sem):
        cid = lax.axis_index("c")
        sid = lax.axis_index("s")
        w = cid * 16 + sid

        def zrow(r, carry):
            for c in range(de // LANES):
                rows[r, pl.ds(c * LANES, LANES)] = jnp.zeros(
                    (LANES,), jnp.float32)
            return carry
        lax.fori_loop(0, KCH, zrow, 0)

        def zacc(j, carry):
            off = pl.multiple_of(sid * rpt + j * nz, 8)
            pltpu.sync_copy(rows.at[pl.ds(0, nz)], acc.at[pl.ds(off, nz)])
            return carry
        lax.fori_loop(0, 6, zacc, 0)

        @pl.when(sid == LANES - 1)
        def _():
            pltpu.sync_copy(rows.at[pl.ds(0, rem)],
                            acc.at[pl.ds(rem_off, rem)])

        pltpu.sync_copy(a_hbm, a_v)
        pltpu.sync_copy(b_hbm, b_v)
        plsc.subcore_barrier()

        def mstep(i, m):
            return jnp.maximum(m, a_v[pl.ds(i * LANES, LANES)])
        m0 = a_v[pl.ds(0, LANES)]
        m = lax.fori_loop(1, n // LANES, mstep, m0)
        ii = lax.iota(jnp.int32, LANES)
        for sft in (8, 4, 2, 1):
            p_v[pl.ds(0, LANES)] = m
            m = jnp.maximum(m, plsc.load_gather(p_v, [ii ^ sft]))
        amax = m

        ebase = w * epw_pad

        def chunk(j, carry):
            base = pl.multiple_of(ebase + j * KCH, KCH)
            pltpu.sync_copy(src_hbm.at[pl.ds(base, KCH)], sidx)
            pltpu.sync_copy(dst_hbm.at[pl.ds(base, KCH)], didx)
            gcopy = pltpu.async_copy(he_hbm.at[sidx], rows, sem)

            for g in range(KCH // LANES):
                sv = sidx[pl.ds(g * LANES, LANES)]
                dv = didx[pl.ds(g * LANES, LANES)]
                av = plsc.load_gather(a_v, [sv])
                bv = plsc.load_gather(b_v, [dv])
                e = _lrelu(av + bv)
                sh = _lrelu(bv + amax)
                pos = j * KCH + g * LANES + lax.iota(jnp.int32, LANES)
                p = jnp.where(pos < epw, jnp.exp(e - sh), 0.0)
                p_v[pl.ds(g * LANES, LANES)] = p

            gcopy.wait()

            def scale(g, carry2):
                for i in range(LANES):
                    kk = g * LANES + i
                    pb = plsc.load_gather(
                        p_v, [jnp.full((LANES,), kk, jnp.int32)])
                    for c in range(de // LANES):
                        sl = pl.ds(c * LANES, LANES)
                        rows[kk, sl] = rows[kk, sl] * pb
                return carry2
            lax.fori_loop(0, KCH // LANES, scale, 0)

            pltpu.sync_copy(rows, acc.at[didx], add=True)
            return carry

        lax.fori_loop(0, nchunks, chunk, 0)
        plsc.subcore_barrier()

        woff = pl.multiple_of(sid * rpt, 8)
        pltpu.sync_copy(acc.at[pl.ds(woff, rpt)],
                        out_hbm.at[cid].at[pl.ds(woff, rpt)])

        @pl.when(sid == LANES - 1)
        def _():
            pltpu.sync_copy(acc.at[pl.ds(rem_off, rem)],
                            out_hbm.at[cid].at[pl.ds(rem_off, rem)])

    return edge_kernel


def kernel(X, edge_index, W1, att1_src, att1_dst, W2, att2_src, att2_dst):
    n, d_in = X.shape
    d_hid = W1.shape[1]
    e = edge_index.shape[1]
    epw = e // NW
    epw_pad = ((epw + KCH - 1) // KCH) * KCH
    pad = epw_pad - epw

    src = edge_index[0].reshape(NW, epw)
    dst = edge_index[1].reshape(NW, epw)
    if pad:
        padv = jnp.broadcast_to(
            jnp.arange(pad, dtype=jnp.int32) % n, (NW, pad))
        src = jnp.concatenate([src, padv], axis=1)
        dst = jnp.concatenate([dst, padv], axis=1)
    src = src.reshape(-1)
    dst = dst.reshape(-1)

    he1, a1, b1, ps1 = _dense_stage(X, W1, att1_src, att1_dst)
    num1 = _edge_pass(n, d_hid, epw, epw_pad)(
        src, dst, a1.reshape(-1), b1.reshape(-1), he1)
    H = _normalize(num1, ps1, he1, d_hid, act=False)

    he2, a2, b2, ps2 = _dense_stage(H, W2, att2_src, att2_dst)
    num2 = _edge_pass(n, d_in, epw, epw_pad)(
        src, dst, a2.reshape(-1), b2.reshape(-1), he2)
    X_rec = _normalize(num2, ps2, he2, d_in, act=True)

    return (H, X_rec)
